# Initial kernel scaffold; baseline (speedup 1.0000x reference)
import jax
import jax.numpy as jnp
from jax.experimental import pallas as pl


def kernel(features, adj, W, active):
    raise NotImplementedError("write your pallas kernel here")


# speedup vs baseline: 1.0397x; 1.0397x over previous
import jax
import jax.numpy as jnp
from jax.experimental import pallas as pl
from jax.experimental.pallas import tpu as pltpu

_N = 10000
_F = 128
_BM = 400


def _gnn_kernel(active_ref, features_ref, w_ref, adj_ref, out_ref, support_ref):
    i = pl.program_id(0)

    @pl.when(i == 0)
    def _():
        s = jnp.dot(features_ref[...], w_ref[...],
                    preferred_element_type=jnp.float32)
        support_ref[...] = jnp.where(active_ref[0] != 0, jnp.tanh(s), s)

    out_ref[...] = jnp.dot(adj_ref[...], support_ref[...],
                           preferred_element_type=jnp.float32)


def kernel(features, adj, W, active):
    active_arr = jnp.asarray(active, jnp.int32).reshape((1,))
    return pl.pallas_call(
        _gnn_kernel,
        grid_spec=pltpu.PrefetchScalarGridSpec(
            num_scalar_prefetch=1,
            grid=(_N // _BM,),
            in_specs=[
                pl.BlockSpec((_N, _F), lambda i, a: (0, 0)),
                pl.BlockSpec((_F, _F), lambda i, a: (0, 0)),
                pl.BlockSpec((_BM, _N), lambda i, a: (i, 0)),
            ],
            out_specs=pl.BlockSpec((_BM, _F), lambda i, a: (i, 0)),
            scratch_shapes=[pltpu.VMEM((_N, _F), jnp.float32)],
        ),
        out_shape=jax.ShapeDtypeStruct((_N, _F), jnp.float32),
        compiler_params=pltpu.CompilerParams(
            dimension_semantics=("arbitrary",),
        ),
    )(active_arr, features, W, adj)
